# baseline (speedup 1.0000x reference)
import jax
import jax.numpy as jnp
from jax import lax
from jax.experimental import pallas as pl
from jax.experimental.pallas import tpu as pltpu
from jax.experimental.pallas import tpu_sc as plsc

_BATCH = 16384
_K = 128
_NC = 2
_NS = 16
_NW = _NC * _NS
_BPW = _BATCH // _NW
_CHUNKS = (64, 128, 160, 160)
_CMAX = max(_CHUNKS)
_L = 16
_NSLOT = 2


def _mirt_body(sid_hbm, eid_hbm, theta_hbm, a_hbm, b_hbm, out_hbm,
               theta_v0, a_v0, theta_v1, a_v1,
               sid_v, eid_v, b_v, part_v, out_v,
               sem0, sem1, semb):
    wid = lax.axis_index("s") * _NC + lax.axis_index("c")
    base = wid * _BPW

    s_cp = pltpu.async_copy(sid_hbm.at[pl.ds(base, _BPW)], sid_v, semb)
    e_cp = pltpu.async_copy(eid_hbm.at[pl.ds(base, _BPW)], eid_v, semb)
    s_cp.wait()
    e_cp.wait()

    slots = ((theta_v0, a_v0, sem0), (theta_v1, a_v1, sem1))
    offs = [sum(_CHUNKS[:i]) for i in range(len(_CHUNKS))]

    def issue(c):
        th_b, a_b, sem = slots[c % _NSLOT]
        off, sz = offs[c], _CHUNKS[c]
        t_cp = pltpu.async_copy(theta_hbm.at[sid_v.at[pl.ds(off, sz)]],
                                th_b.at[pl.ds(0, sz)], sem)
        a_cp = pltpu.async_copy(a_hbm.at[eid_v.at[pl.ds(off, sz)]],
                                a_b.at[pl.ds(0, sz)], sem)
        return t_cp, a_cp

    pending = [None] * _NSLOT
    pending[0] = issue(0)
    b_cp = pltpu.async_copy(b_hbm.at[eid_v], b_v, semb)

    for c in range(len(_CHUNKS)):
        if c + 1 < len(_CHUNKS):
            pending[(c + 1) % _NSLOT] = issue(c + 1)
        t_cp, a_cp = pending[c % _NSLOT]
        t_cp.wait()
        a_cp.wait()
        th_b, a_b, _ = slots[c % _NSLOT]
        off = offs[c]

        @plsc.parallel_loop(0, _CHUNKS[c], unroll=4)
        def _item(i, th_b=th_b, a_b=a_b, off=off):
            acc = jnp.zeros((_L,), jnp.float32)
            for k in range(_K // _L):
                av = a_b[i, pl.ds(k * _L, _L)]
                tv = th_b[i, pl.ds(k * _L, _L)]
                acc = acc + tv / (1.0 + jnp.exp(-av))
            part_v[pl.ds((off + i) * _L, _L)] = acc

    b_cp.wait()
    lanes = lax.iota(jnp.int32, _L)

    @plsc.parallel_loop(0, _BPW // _L, unroll=2)
    def _group(g):
        row0 = g * _L
        flat0 = row0 * _L + lanes * _L
        acc = jnp.zeros((_L,), jnp.float32)
        for j in range(_L):
            acc = acc + plsc.load_gather(part_v, [flat0 + j])
        bv = b_v[pl.ds(row0, _L)]
        out_v[pl.ds(row0, _L)] = 1.0 / (1.0 + jnp.exp(bv - acc))

    pltpu.sync_copy(out_v, out_hbm.at[pl.ds(base, _BPW)])


def _mirt_call(student_id, exercise_id, theta_w, a_w, b_flat, interpret=False):
    mesh = plsc.VectorSubcoreMesh(core_axis_name="c", subcore_axis_name="s",
                                  num_cores=_NC, num_subcores=_NS)
    chunk_slot = [
        pltpu.VMEM((_CMAX, _K), jnp.float32),
        pltpu.VMEM((_CMAX, _K), jnp.float32),
    ]
    run = pl.kernel(
        _mirt_body,
        out_type=jax.ShapeDtypeStruct((_BATCH,), jnp.float32),
        mesh=mesh,
        scratch_types=chunk_slot + chunk_slot + [
            pltpu.VMEM((_BPW,), jnp.int32),
            pltpu.VMEM((_BPW,), jnp.int32),
            pltpu.VMEM((_BPW,), jnp.float32),
            pltpu.VMEM((_BPW * _L,), jnp.float32),
            pltpu.VMEM((_BPW,), jnp.float32),
            pltpu.SemaphoreType.DMA,
            pltpu.SemaphoreType.DMA,
            pltpu.SemaphoreType.DMA,
        ],
        compiler_params=pltpu.CompilerParams(
            needs_layout_passes=False,
            disable_bounds_checks=True,
            disable_semaphore_checks=True,
        ),
        interpret=interpret,
    )
    return run(student_id, exercise_id, theta_w, a_w, b_flat)


def kernel(student_id, exercise_id, theta_w, a_w, b_w):
    return _mirt_call(student_id, exercise_id, theta_w, a_w,
                      b_w.reshape((-1,)))


# speedup vs baseline: 1.0160x; 1.0160x over previous
import jax
import jax.numpy as jnp
from jax import lax
from jax.experimental import pallas as pl
from jax.experimental.pallas import tpu as pltpu
from jax.experimental.pallas import tpu_sc as plsc

_BATCH = 16384
_K = 128
_NC = 2
_NS = 16
_NW = _NC * _NS
_BPW = _BATCH // _NW
_CHUNKS = (32, 96, 128, 128, 128)
_CMAX = max(_CHUNKS)
_L = 16
_NSLOT = 2


def _mirt_body(sid_hbm, eid_hbm, theta_hbm, a_hbm, b_hbm, out_hbm,
               theta_v0, a_v0, theta_v1, a_v1,
               sid_v, eid_v, b_v, part_v, out_v,
               sem0, sem1, semb):
    wid = lax.axis_index("s") * _NC + lax.axis_index("c")
    base = wid * _BPW

    s_cp = pltpu.async_copy(sid_hbm.at[pl.ds(base, _BPW)], sid_v, semb)
    e_cp = pltpu.async_copy(eid_hbm.at[pl.ds(base, _BPW)], eid_v, semb)
    s_cp.wait()
    e_cp.wait()

    slots = ((theta_v0, a_v0, sem0), (theta_v1, a_v1, sem1))
    offs = [sum(_CHUNKS[:i]) for i in range(len(_CHUNKS))]

    def issue(c):
        th_b, a_b, sem = slots[c % _NSLOT]
        off, sz = offs[c], _CHUNKS[c]
        t_cp = pltpu.async_copy(theta_hbm.at[sid_v.at[pl.ds(off, sz)]],
                                th_b.at[pl.ds(0, sz)], sem)
        a_cp = pltpu.async_copy(a_hbm.at[eid_v.at[pl.ds(off, sz)]],
                                a_b.at[pl.ds(0, sz)], sem)
        return t_cp, a_cp

    pending = [None] * _NSLOT
    pending[0] = issue(0)
    b_cp = pltpu.async_copy(b_hbm.at[eid_v], b_v, semb)

    for c in range(len(_CHUNKS)):
        if c + 1 < len(_CHUNKS):
            pending[(c + 1) % _NSLOT] = issue(c + 1)
        t_cp, a_cp = pending[c % _NSLOT]
        t_cp.wait()
        a_cp.wait()
        th_b, a_b, _ = slots[c % _NSLOT]
        off = offs[c]

        @plsc.parallel_loop(0, _CHUNKS[c], unroll=4)
        def _item(i, th_b=th_b, a_b=a_b, off=off):
            acc = jnp.zeros((_L,), jnp.float32)
            for k in range(_K // _L):
                av = a_b[i, pl.ds(k * _L, _L)]
                tv = th_b[i, pl.ds(k * _L, _L)]
                acc = acc + tv / (1.0 + jnp.exp(-av))
            part_v[pl.ds((off + i) * _L, _L)] = acc

    b_cp.wait()
    lanes = lax.iota(jnp.int32, _L)

    @plsc.parallel_loop(0, _BPW // _L, unroll=2)
    def _group(g):
        row0 = g * _L
        flat0 = row0 * _L + lanes * _L
        acc = jnp.zeros((_L,), jnp.float32)
        for j in range(_L):
            acc = acc + plsc.load_gather(part_v, [flat0 + j])
        bv = b_v[pl.ds(row0, _L)]
        out_v[pl.ds(row0, _L)] = 1.0 / (1.0 + jnp.exp(bv - acc))

    pltpu.sync_copy(out_v, out_hbm.at[pl.ds(base, _BPW)])


def _mirt_call(student_id, exercise_id, theta_w, a_w, b_flat, interpret=False):
    mesh = plsc.VectorSubcoreMesh(core_axis_name="c", subcore_axis_name="s",
                                  num_cores=_NC, num_subcores=_NS)
    chunk_slot = [
        pltpu.VMEM((_CMAX, _K), jnp.float32),
        pltpu.VMEM((_CMAX, _K), jnp.float32),
    ]
    run = pl.kernel(
        _mirt_body,
        out_type=jax.ShapeDtypeStruct((_BATCH,), jnp.float32),
        mesh=mesh,
        scratch_types=chunk_slot + chunk_slot + [
            pltpu.VMEM((_BPW,), jnp.int32),
            pltpu.VMEM((_BPW,), jnp.int32),
            pltpu.VMEM((_BPW,), jnp.float32),
            pltpu.VMEM((_BPW * _L,), jnp.float32),
            pltpu.VMEM((_BPW,), jnp.float32),
            pltpu.SemaphoreType.DMA,
            pltpu.SemaphoreType.DMA,
            pltpu.SemaphoreType.DMA,
        ],
        compiler_params=pltpu.CompilerParams(needs_layout_passes=False),
        interpret=interpret,
    )
    return run(student_id, exercise_id, theta_w, a_w, b_flat)


def kernel(student_id, exercise_id, theta_w, a_w, b_w):
    return _mirt_call(student_id, exercise_id, theta_w, a_w,
                      b_w.reshape((-1,)))
